# baseline (speedup 1.0000x reference)
import functools
import math

import jax
import jax.numpy as jnp
from jax import lax
from jax.experimental import pallas as pl
from jax.experimental.pallas import tpu as pltpu
from jax.experimental.pallas import tpu_sc as plsc

D_MODEL = 64
SCALE = math.sqrt(D_MODEL)
LANES = 16

NUM_CORES = 2
NUM_SUBCORES = 16
NUM_WORKERS = NUM_CORES * NUM_SUBCORES

CHUNK = 100


@functools.partial(jax.jit, static_argnames=("b", "t"))
def _embed_sc(x3d, weight, *, b, t):
    b_per_w = b // NUM_WORKERS
    n_chunks = b_per_w * (t // CHUNK)
    mesh = plsc.VectorSubcoreMesh(core_axis_name="c", subcore_axis_name="s")

    @functools.partial(
        pl.kernel,
        out_type=jax.ShapeDtypeStruct((b, t, D_MODEL), jnp.float32),
        mesh=mesh,
        scratch_types=[
            pltpu.VMEM((n_chunks, CHUNK), jnp.int32),
            pltpu.VMEM((CHUNK, D_MODEL), jnp.float32),
            pltpu.SemaphoreType.DMA,
        ],
        compiler_params=pltpu.CompilerParams(use_tc_tiling_on_sc=False),
    )
    def body(w_hbm, idx_hbm, out_hbm, idx_v, rows_v, gsem):
        wid = lax.axis_index("s") * NUM_CORES + lax.axis_index("c")
        base_b = wid * b_per_w
        pltpu.sync_copy(idx_hbm.at[wid], idx_v)

        @pl.loop(0, n_chunks)
        def chunk_loop(c):
            pltpu.async_copy(w_hbm.at[idx_v.at[c]], rows_v, gsem).wait()

            @pl.loop(0, CHUNK)
            def row_loop(i):
                for j in range(D_MODEL // LANES):
                    sl = pl.ds(j * LANES, LANES)
                    rows_v[i, sl] = rows_v[i, sl] * SCALE

            pltpu.sync_copy(
                rows_v,
                out_hbm.at[base_b + c // 2, pl.ds((c % 2) * CHUNK, CHUNK)],
            )

    return body(weight, x3d)


def kernel(x, weight):
    b, t = x.shape
    assert b % NUM_WORKERS == 0 and t % CHUNK == 0
    n_chunks = (b // NUM_WORKERS) * (t // CHUNK)
    x3d = x.reshape(NUM_WORKERS, n_chunks, CHUNK).astype(jnp.int32)
    return _embed_sc(x3d, weight, b=b, t=t)
